# baseline (speedup 1.0000x reference)
import functools

import jax
import jax.numpy as jnp
from jax import lax
from jax.experimental import pallas as pl
from jax.experimental.pallas import tpu as pltpu
from jax.experimental.pallas import tpu_sc as plsc

N_NODES = 10000
N_EDGES = 160000
N_EPAD = 172032
NB = 5
NGRAPH = 4
ROW_BLK = 2000
EDGE_BLK = 4096
NEG_BIG = -3.0e38


def _mm_kernel(x_ref, w_ref, b_ref, o_ref, *, act):
    y = jnp.dot(x_ref[...], w_ref[...], preferred_element_type=jnp.float32)
    y = y + b_ref[...]
    if act:
        y = jnp.maximum(y, 0.0)
    o_ref[...] = y


def _mm(x, w, b, act=True, blk=ROW_BLK):
    m, k = x.shape
    n = w.shape[1]
    return pl.pallas_call(
        functools.partial(_mm_kernel, act=act),
        grid=(m // blk,),
        in_specs=[
            pl.BlockSpec((blk, k), lambda i: (i, 0)),
            pl.BlockSpec((k, n), lambda i: (0, 0)),
            pl.BlockSpec((1, n), lambda i: (0, 0)),
        ],
        out_specs=pl.BlockSpec((blk, n), lambda i: (i, 0)),
        out_shape=jax.ShapeDtypeStruct((m, n), jnp.float32),
    )(x, w, b.reshape(1, n))


def _mm2_kernel(x1_ref, x2_ref, w1_ref, w2_ref, b_ref, o_ref, *, act):
    y = jnp.dot(x1_ref[...], w1_ref[...], preferred_element_type=jnp.float32)
    y = y + jnp.dot(x2_ref[...], w2_ref[...], preferred_element_type=jnp.float32)
    y = y + b_ref[...]
    if act:
        y = jnp.maximum(y, 0.0)
    o_ref[...] = y


def _mm2(x1, x2, w1, w2, b, act=True, blk=ROW_BLK):
    m, k1 = x1.shape
    k2 = x2.shape[1]
    n = w1.shape[1]
    return pl.pallas_call(
        functools.partial(_mm2_kernel, act=act),
        grid=(m // blk,),
        in_specs=[
            pl.BlockSpec((blk, k1), lambda i: (i, 0)),
            pl.BlockSpec((blk, k2), lambda i: (i, 0)),
            pl.BlockSpec((k1, n), lambda i: (0, 0)),
            pl.BlockSpec((k2, n), lambda i: (0, 0)),
            pl.BlockSpec((1, n), lambda i: (0, 0)),
        ],
        out_specs=pl.BlockSpec((blk, n), lambda i: (i, 0)),
        out_shape=jax.ShapeDtypeStruct((m, n), jnp.float32),
    )(x1, x2, w1, w2, b.reshape(1, n))


def _gcu_ab_kernel(raw_ref, feat_ref, wp_ref, bp_ref, wt_ref, wb_ref, bc_ref,
                   *o_refs):
    pf = jnp.dot(raw_ref[...], wp_ref[...], preferred_element_type=jnp.float32)
    pf = jnp.maximum(pf + bp_ref[...], 0.0)
    y = jnp.dot(pf, wt_ref[...], preferred_element_type=jnp.float32)
    y = y + jnp.dot(feat_ref[...], wb_ref[...], preferred_element_type=jnp.float32)
    y = y + bc_ref[...]
    for t, o in enumerate(o_refs):
        o[...] = y[:, t * 128:(t + 1) * 128]


def _gcu_ab(raw, feat, wp, bp, wtop, wbot, bcat):
    m = raw.shape[0]
    kr = raw.shape[1]
    kf = feat.shape[1]
    outs = pl.pallas_call(
        _gcu_ab_kernel,
        grid=(m // ROW_BLK,),
        in_specs=[
            pl.BlockSpec((ROW_BLK, kr), lambda i: (i, 0)),
            pl.BlockSpec((ROW_BLK, kf), lambda i: (i, 0)),
            pl.BlockSpec((kr, 64), lambda i: (0, 0)),
            pl.BlockSpec((1, 64), lambda i: (0, 0)),
            pl.BlockSpec((64, 512), lambda i: (0, 0)),
            pl.BlockSpec((kf, 512), lambda i: (0, 0)),
            pl.BlockSpec((1, 512), lambda i: (0, 0)),
        ],
        out_specs=[pl.BlockSpec((ROW_BLK, 128), lambda i: (i, 0))] * 4,
        out_shape=[jax.ShapeDtypeStruct((m, 128), jnp.float32)] * 4,
    )(raw, feat, wp, bp.reshape(1, 64), wtop, wbot, bcat.reshape(1, 512))
    return outs


def _edge_mm_kernel(g_ref, w_ref, b_ref, o_ref):
    y = jnp.dot(g_ref[...], w_ref[...], preferred_element_type=jnp.float32)
    o_ref[...] = jnp.maximum(y + b_ref[...], 0.0)


def _edge_mm(g, w, b):
    e = g.shape[0]
    return pl.pallas_call(
        _edge_mm_kernel,
        grid=(e // EDGE_BLK,),
        in_specs=[
            pl.BlockSpec((EDGE_BLK, 128), lambda i: (i, 0)),
            pl.BlockSpec((128, 128), lambda i: (0, 0)),
            pl.BlockSpec((1, 128), lambda i: (0, 0)),
        ],
        out_specs=pl.BlockSpec((EDGE_BLK, 128), lambda i: (i, 0)),
        out_shape=jax.ShapeDtypeStruct((e, 128), jnp.float32),
    )(g, w, b.reshape(1, 128))


def _mlt2_segmax_kernel(x_ref, w_ref, b_ref, batch_ref, o_ref, acc_ref):
    i = pl.program_id(0)

    @pl.when(i == 0)
    def _():
        acc_ref[...] = jnp.full_like(acc_ref, NEG_BIG)

    y = jnp.dot(x_ref[...], w_ref[...], preferred_element_type=jnp.float32)
    y = jnp.maximum(y + b_ref[...], 0.0)
    bids = batch_ref[...]
    for g in range(NGRAPH):
        m = (bids == g)
        colmax = jnp.max(jnp.where(m, y, NEG_BIG), axis=0)
        acc_ref[g, :] = jnp.maximum(acc_ref[g, :], colmax)

    @pl.when(i == pl.num_programs(0) - 1)
    def _():
        a = acc_ref[...]
        o_ref[...] = jnp.where(a <= NEG_BIG, 0.0, a)


def _mlt2_segmax(x, w, b, batch):
    m, k = x.shape
    n = w.shape[1]
    return pl.pallas_call(
        _mlt2_segmax_kernel,
        grid=(m // ROW_BLK,),
        in_specs=[
            pl.BlockSpec((ROW_BLK, k), lambda i: (i, 0)),
            pl.BlockSpec((k, n), lambda i: (0, 0)),
            pl.BlockSpec((1, n), lambda i: (0, 0)),
            pl.BlockSpec((ROW_BLK, 1), lambda i: (i, 0)),
        ],
        out_specs=pl.BlockSpec((8, n), lambda i: (0, 0)),
        out_shape=jax.ShapeDtypeStruct((8, n), jnp.float32),
        scratch_shapes=[pltpu.VMEM((8, n), jnp.float32)],
    )(x, w, b.reshape(1, n), batch.reshape(m, 1))


def _cls_kernel(x3_ref, xg4_ref, batch_ref, wt_ref, wb_ref, b1_ref,
                w2_ref, b2_ref, w3_ref, b3_ref, o_ref):
    xgw = jnp.dot(xg4_ref[...], wb_ref[...], preferred_element_type=jnp.float32)
    bids = batch_ref[...]
    onehot = (bids == lax.broadcasted_iota(jnp.int32, (1, 8), 1)).astype(jnp.float32)
    gathered = jnp.dot(onehot, xgw, preferred_element_type=jnp.float32)
    h = jnp.dot(x3_ref[...], wt_ref[...], preferred_element_type=jnp.float32)
    h = jnp.maximum(h + gathered + b1_ref[...], 0.0)
    h = jnp.dot(h, w2_ref[...], preferred_element_type=jnp.float32)
    h = jnp.maximum(h + b2_ref[...], 0.0)
    y = jnp.dot(h, w3_ref[...], preferred_element_type=jnp.float32)
    o_ref[...] = y + b3_ref[...]


def _cls_head(x3, xg4, batch, wtop, wbot, b1, w2, b2, w3, b3):
    m = x3.shape[0]
    nout = w3.shape[1]
    return pl.pallas_call(
        _cls_kernel,
        grid=(m // ROW_BLK,),
        in_specs=[
            pl.BlockSpec((ROW_BLK, 256), lambda i: (i, 0)),
            pl.BlockSpec((8, 1024), lambda i: (0, 0)),
            pl.BlockSpec((ROW_BLK, 1), lambda i: (i, 0)),
            pl.BlockSpec((256, 1024), lambda i: (0, 0)),
            pl.BlockSpec((1024, 1024), lambda i: (0, 0)),
            pl.BlockSpec((1, 1024), lambda i: (0, 0)),
            pl.BlockSpec((1024, 512), lambda i: (0, 0)),
            pl.BlockSpec((1, 512), lambda i: (0, 0)),
            pl.BlockSpec((512, nout), lambda i: (0, 0)),
            pl.BlockSpec((1, nout), lambda i: (0, 0)),
        ],
        out_specs=pl.BlockSpec((ROW_BLK, nout), lambda i: (i, 0)),
        out_shape=jax.ShapeDtypeStruct((m, nout), jnp.float32),
    )(x3, xg4, batch.reshape(m, 1), wtop, wbot, b1.reshape(1, 1024),
      w2, b2.reshape(1, 512), w3, b3.reshape(1, nout))


_NC, _NS = 2, 16
_NW = _NC * _NS
_EPW = N_EPAD // _NW
_GCH = 896
_GH = _GCH // 2
_DN = 313
_DN1 = _DN + 1
_NP = _NW * _DN
_SCH = 4000
_GB = 128
_CBUF = _SCH + _GB + 32
_CB2 = _SCH + _GB
_CAP = 168192
_NCHK = N_EDGES // _SCH


def _sc_mesh():
    return plsc.VectorSubcoreMesh(core_axis_name="c", subcore_axis_name="s")


_GQ = _GCH // 4


def _sc_gather_add(a_tab, b_tab, dst, src):
    @functools.partial(
        pl.kernel, mesh=_sc_mesh(),
        compiler_params=pltpu.CompilerParams(needs_layout_passes=False),
        out_type=jax.ShapeDtypeStruct((N_EPAD, 128), jnp.float32),
        scratch_types=[
            pltpu.VMEM((_GCH,), jnp.int32),
            pltpu.VMEM((_GCH,), jnp.int32),
            pltpu.VMEM((_GQ, 128), jnp.float32),
            pltpu.VMEM((_GQ, 128), jnp.float32),
            pltpu.VMEM((_GQ, 128), jnp.float32),
            pltpu.VMEM((_GQ, 128), jnp.float32),
            pltpu.SemaphoreType.DMA,
            pltpu.SemaphoreType.DMA,
            pltpu.SemaphoreType.DMA,
            pltpu.SemaphoreType.DMA,
            pltpu.SemaphoreType.DMA,
            pltpu.SemaphoreType.DMA,
        ])
    def k(a_hbm, b_hbm, dst_hbm, src_hbm, g_hbm,
          di_v, si_v, a0, a1, b0, b1, ga0, ga1, gb0, gb1, w0, w1):
        wid = lax.axis_index("s") * _NC + lax.axis_index("c")
        base = wid * _EPW
        abufs, bbufs = (a0, a1), (b0, b1)
        gas, gbs, ws = (ga0, ga1), (gb0, gb1), (w0, w1)

        def body(c, carry):
            off = base + c * _GCH
            pltpu.sync_copy(dst_hbm.at[pl.ds(off, _GCH)], di_v)
            pltpu.sync_copy(src_hbm.at[pl.ds(off, _GCH)], si_v)

            def startg(q):
                p = q % 2
                pltpu.async_copy(a_hbm.at[di_v.at[pl.ds(q * _GQ, _GQ)]],
                                 abufs[p], gas[p])
                pltpu.async_copy(b_hbm.at[si_v.at[pl.ds(q * _GQ, _GQ)]],
                                 bbufs[p], gbs[p])

            startg(0)
            startg(1)
            for q in range(4):
                p = q % 2
                pltpu.make_async_copy(a_hbm.at[di_v.at[pl.ds(q * _GQ, _GQ)]],
                                      abufs[p], gas[p]).wait()
                pltpu.make_async_copy(b_hbm.at[si_v.at[pl.ds(q * _GQ, _GQ)]],
                                      bbufs[p], gbs[p]).wait()

                def fuse(r, cc):
                    for kk in range(8):
                        sl = pl.ds(kk * 16, 16)
                        av = abufs[p][r, sl]
                        bv = bbufs[p][r, sl]
                        abufs[p][r, sl] = jnp.maximum(av + bv, 0.0)
                    return cc

                lax.fori_loop(0, _GQ, fuse, 0)
                pltpu.async_copy(abufs[p],
                                 g_hbm.at[pl.ds(off + q * _GQ, _GQ)], ws[p])
                if q + 2 < 4:
                    pltpu.make_async_copy(
                        abufs[p], g_hbm.at[pl.ds(off + q * _GQ, _GQ)],
                        ws[p]).wait()
                    startg(q + 2)
            for q in (2, 3):
                p = q % 2
                pltpu.make_async_copy(
                    abufs[p], g_hbm.at[pl.ds(off + q * _GQ, _GQ)],
                    ws[p]).wait()
            return carry

        lax.fori_loop(0, _EPW // _GCH, body, 0)

    return k(a_tab, b_tab, dst, src)


def _sc_compact(dst):
    @functools.partial(
        pl.kernel, mesh=_sc_mesh(),
        compiler_params=pltpu.CompilerParams(needs_layout_passes=False),
        out_type=[jax.ShapeDtypeStruct((_NW * _CAP,), jnp.int32),
                  jax.ShapeDtypeStruct((_NW * _CAP,), jnp.int32),
                  jax.ShapeDtypeStruct((_NW * 16,), jnp.int32)],
        scratch_types=[
            pltpu.VMEM((_SCH,), jnp.int32),
            pltpu.VMEM((_CBUF,), jnp.int32),
            pltpu.VMEM((_CBUF,), jnp.int32),
            pltpu.VMEM((_CBUF,), jnp.int32),
            pltpu.VMEM((_CBUF,), jnp.int32),
            pltpu.VMEM((16,), jnp.int32),
            pltpu.SemaphoreType.DMA,
            pltpu.SemaphoreType.DMA,
            pltpu.SemaphoreType.DMA,
            pltpu.SemaphoreType.DMA,
        ])
    def k(dst_hbm, ce_hbm, cl_hbm, cnt_hbm,
          dbuf, ce0, ce1, cl0, cl1, cbuf, se0, se1, sl0, sl1):
        wid = lax.axis_index("s") * _NC + lax.axis_index("c")
        lo = wid * _DN
        base = wid * _CAP
        lane = lax.iota(jnp.int32, 16)
        dn_splat = jnp.full((16,), _DN, jnp.int32)
        zero_splat = jnp.full((16,), 0, jnp.int32)
        cebufs, clbufs = (ce0, ce1), (cl0, cl1)
        esems, lsems = (se0, se1), (sl0, sl1)

        def one_chunk(c, u, cg):
            pltpu.sync_copy(dst_hbm.at[pl.ds(c * _SCH, _SCH)], dbuf)

            def scan(j, ptr):
                vd = dbuf[pl.ds(j * 16, 16)]
                m = (vd >= lo) & (vd < lo + _DN)
                mi = jnp.where(m, 1, 0)
                pos = plsc.cumsum(mi)
                eid = c * _SCH + j * 16 + lane
                tgt = jnp.where(m, ptr + pos - 1, _CBUF - 1)
                plsc.store_scatter(cebufs[u], [tgt], eid)
                plsc.store_scatter(clbufs[u], [tgt], vd - lo)
                return ptr + jnp.max(pos)

            total = lax.fori_loop(0, _SCH // 16, scan, 0)
            cebufs[u][pl.ds(total, 16)] = zero_splat
            clbufs[u][pl.ds(total, 16)] = dn_splat
            pltpu.async_copy(cebufs[u].at[pl.ds(0, _SCH)],
                             ce_hbm.at[pl.ds(base + cg * 8, _SCH)], esems[u])
            pltpu.async_copy(clbufs[u].at[pl.ds(0, _SCH)],
                             cl_hbm.at[pl.ds(base + cg * 8, _SCH)], lsems[u])
            return cg + (total + 7) // 8

        def body(i, cg):
            for u in range(2):
                @pl.when(i > 0)
                def _():
                    pltpu.make_async_copy(
                        cebufs[u].at[pl.ds(0, _SCH)],
                        ce_hbm.at[pl.ds(base, _SCH)], esems[u]).wait()
                    pltpu.make_async_copy(
                        clbufs[u].at[pl.ds(0, _SCH)],
                        cl_hbm.at[pl.ds(base, _SCH)], lsems[u]).wait()
                cg = one_chunk(2 * i + u, u, cg)
            return cg

        cum = lax.fori_loop(0, _NCHK // 2, body, 0) * 8
        for u in range(2):
            pltpu.make_async_copy(cebufs[u].at[pl.ds(0, _SCH)],
                                  ce_hbm.at[pl.ds(base, _SCH)], esems[u]).wait()
            pltpu.make_async_copy(clbufs[u].at[pl.ds(0, _SCH)],
                                  cl_hbm.at[pl.ds(base, _SCH)], lsems[u]).wait()
        cbuf[pl.ds(0, 16)] = zero_splat + cum
        pltpu.sync_copy(cbuf, cnt_hbm.at[pl.ds(wid * 16, 16)])

    return k(dst)


def _sc_segmax(h, comp):
    ce_all, cl_all, cnts = comp
    zeros_f = jnp.zeros((_DN1 * 128,), jnp.float32)

    @functools.partial(
        pl.kernel, mesh=_sc_mesh(),
        compiler_params=pltpu.CompilerParams(needs_layout_passes=False),
        out_type=jax.ShapeDtypeStruct((_NP * 128,), jnp.float32),
        scratch_types=[
            pltpu.VMEM((_DN1 * 128,), jnp.float32),
            pltpu.VMEM((_CB2,), jnp.int32),
            pltpu.VMEM((_CB2,), jnp.int32),
            pltpu.VMEM((16,), jnp.int32),
            pltpu.VMEM((_GB, 128), jnp.float32),
            pltpu.VMEM((_GB, 128), jnp.float32),
            pltpu.SemaphoreType.DMA,
            pltpu.SemaphoreType.DMA,
        ])
    def k(h_hbm, ce_hbm, cl_hbm, cnt_hbm, z_hbm, out_hbm,
          out_l, cebuf, clbuf, cbuf, hbuf0, hbuf1, sem0, sem1):
        wid = lax.axis_index("s") * _NC + lax.axis_index("c")
        lo = wid * _DN
        base = wid * _CAP
        lane = lax.iota(jnp.int32, 16)

        pltpu.sync_copy(z_hbm, out_l)
        pltpu.sync_copy(cnt_hbm.at[pl.ds(wid * 16, 16)], cbuf)
        cnt = jnp.max(cbuf[pl.ds(0, 16)])
        nch = (cnt + _SCH - 1) // _SCH

        def chunk(cix, c0):
            pltpu.sync_copy(ce_hbm.at[pl.ds(base + cix * _SCH, _SCH)],
                            cebuf.at[pl.ds(0, _SCH)])
            pltpu.sync_copy(cl_hbm.at[pl.ds(base + cix * _SCH, _SCH)],
                            clbuf.at[pl.ds(0, _SCH)])
            valid = jnp.minimum(cnt - cix * _SCH, _SCH)

            def mask(j, cc):
                idx = j * 16 + lane
                mm = idx < valid
                ce = cebuf[pl.ds(j * 16, 16)]
                cl = clbuf[pl.ds(j * 16, 16)]
                cebuf[pl.ds(j * 16, 16)] = jnp.where(mm, ce, 0)
                clbuf[pl.ds(j * 16, 16)] = jnp.where(mm, cl, _DN)
                return cc

            lax.fori_loop(0, _CB2 // 16, mask, 0)
            nsub = (valid + _GB - 1) // _GB

            def start(s, buf, sem):
                pltpu.async_copy(h_hbm.at[cebuf.at[pl.ds(s * _GB, _GB)]],
                                 buf, sem)

            def rmw(s, buf, sem):
                pltpu.make_async_copy(h_hbm.at[cebuf.at[pl.ds(s * _GB, _GB)]],
                                      buf, sem).wait()
                for gq in range(_GB // 16):
                    dvec = clbuf[pl.ds(s * _GB + gq * 16, 16)]
                    for i in range(16):
                        dd = jnp.max(jnp.where(lane == i, dvec, 0))
                        rbase = dd * 128
                        for kk in range(8):
                            sl = pl.ds(rbase + kk * 16, 16)
                            hv = buf[gq * 16 + i, pl.ds(kk * 16, 16)]
                            out_l[sl] = jnp.maximum(out_l[sl], hv)

            @pl.when(nsub > 0)
            def _():
                start(0, hbuf0, sem0)

            def sub2(i, c1):
                s0 = 2 * i

                @pl.when(s0 + 1 < nsub)
                def _():
                    start(s0 + 1, hbuf1, sem1)

                rmw(s0, hbuf0, sem0)

                @pl.when(s0 + 2 < nsub)
                def _():
                    start(s0 + 2, hbuf0, sem0)

                @pl.when(s0 + 1 < nsub)
                def _():
                    rmw(s0 + 1, hbuf1, sem1)

                return c1

            lax.fori_loop(0, (nsub + 1) // 2, sub2, 0)
            return c0

        lax.fori_loop(0, nch, chunk, 0)
        pltpu.sync_copy(out_l.at[pl.ds(0, _DN * 128)],
                        out_hbm.at[pl.ds(lo * 128, _DN * 128)])

    out = k(h, ce_all, cl_all, cnts, zeros_f)
    return out.reshape(_NP, 128)[:N_NODES]


def _edge_conv(a_tab, b_tab, w2, b2, dst, src, comp):
    g = _sc_gather_add(a_tab, b_tab, dst, src)
    h = _edge_mm(g, w2, b2)
    return _sc_segmax(h, comp)


def _split_edge_params(gcu_p, xdim):
    out = {}
    for et in ('tpl', 'geo'):
        (w1, b1), (w2, b2) = gcu_p[et]
        w1a, w1b = w1[:xdim], w1[xdim:]
        out[et] = (w1a - w1b, w1b, b1, w2, b2)
    return out


def _gcu_layer(gcu_p, raw, feat, dst_t, src_t, dst_g, src_g, comp_t, comp_g):
    xdim = 64 + feat.shape[1]
    ep = _split_edge_params(gcu_p, xdim)
    (wp, bp), = gcu_p['pos_mlp']
    wtop = jnp.concatenate([ep['tpl'][0][:64], ep['tpl'][1][:64],
                            ep['geo'][0][:64], ep['geo'][1][:64]], axis=1)
    wbot = jnp.concatenate([ep['tpl'][0][64:], ep['tpl'][1][64:],
                            ep['geo'][0][64:], ep['geo'][1][64:]], axis=1)
    zeros = jnp.zeros_like(ep['tpl'][2])
    bcat = jnp.concatenate([ep['tpl'][2], zeros, ep['geo'][2], zeros])
    a_t, b_t, a_g, b_g = _gcu_ab(raw, feat, wp, bp, wtop, wbot, bcat)
    xt = _edge_conv(a_t, b_t, ep['tpl'][3], ep['tpl'][4], dst_t, src_t, comp_t)
    xg = _edge_conv(a_g, b_g, ep['geo'][3], ep['geo'][4], dst_g, src_g, comp_g)
    (wm, bm), = gcu_p['mlp']
    return _mm2(xt, xg, wm[:128], wm[128:], bm)


def kernel(pos, skin_input, motion, tpl_edge_index, geo_edge_index, batch, params):
    raw = jnp.concatenate([pos, skin_input[:, :8 * NB]], axis=1)
    pad = (jnp.arange(N_EPAD - N_EDGES, dtype=jnp.int32) * 13) % N_NODES
    def _p(v):
        return jnp.concatenate([v, pad])
    src_t, dst_t = _p(tpl_edge_index[0]), _p(tpl_edge_index[1])
    src_g, dst_g = _p(geo_edge_index[0]), _p(geo_edge_index[1])
    comp_t = _sc_compact(dst_t)
    comp_g = _sc_compact(dst_g)

    x1 = _gcu_layer(params['gcu1'], raw, motion, dst_t, src_t, dst_g, src_g,
                    comp_t, comp_g)

    (wl1, bl1), (wl2, bl2) = params['mlt2']
    h = _mm(x1, wl1, bl1)
    xg4 = _mlt2_segmax(h, wl2, bl2, batch)

    x2 = _gcu_layer(params['gcu2'], raw, x1, dst_t, src_t, dst_g, src_g,
                    comp_t, comp_g)
    x3 = _gcu_layer(params['gcu3'], raw, x2, dst_t, src_t, dst_g, src_g,
                    comp_t, comp_g)

    (wc1, bc1), (wc2, bc2) = params['cls_mlp']
    wo, bo = params['cls_out']
    return _cls_head(x3, xg4, batch, wc1[:256], wc1[256:], bc1,
                     wc2, bc2, wo, bo)


# speedup vs baseline: 1.3236x; 1.3236x over previous
import functools

import jax
import jax.numpy as jnp
from jax import lax
from jax.experimental import pallas as pl
from jax.experimental.pallas import tpu as pltpu
from jax.experimental.pallas import tpu_sc as plsc

N_NODES = 10000
N_EDGES = 160000
N_EPAD = 172032
NB = 5
NGRAPH = 4
ROW_BLK = 2000
EDGE_BLK = 4096
NEG_BIG = -3.0e38


def _mm_kernel(x_ref, w_ref, b_ref, o_ref, *, act):
    y = jnp.dot(x_ref[...], w_ref[...], preferred_element_type=jnp.float32)
    y = y + b_ref[...]
    if act:
        y = jnp.maximum(y, 0.0)
    o_ref[...] = y


def _mm(x, w, b, act=True, blk=ROW_BLK):
    m, k = x.shape
    n = w.shape[1]
    return pl.pallas_call(
        functools.partial(_mm_kernel, act=act),
        grid=(m // blk,),
        in_specs=[
            pl.BlockSpec((blk, k), lambda i: (i, 0)),
            pl.BlockSpec((k, n), lambda i: (0, 0)),
            pl.BlockSpec((1, n), lambda i: (0, 0)),
        ],
        out_specs=pl.BlockSpec((blk, n), lambda i: (i, 0)),
        out_shape=jax.ShapeDtypeStruct((m, n), jnp.float32),
    )(x, w, b.reshape(1, n))


def _mm2_kernel(x1_ref, x2_ref, w1_ref, w2_ref, b_ref, o_ref, *, act):
    y = jnp.dot(x1_ref[...], w1_ref[...], preferred_element_type=jnp.float32)
    y = y + jnp.dot(x2_ref[...], w2_ref[...], preferred_element_type=jnp.float32)
    y = y + b_ref[...]
    if act:
        y = jnp.maximum(y, 0.0)
    o_ref[...] = y


def _mm2(x1, x2, w1, w2, b, act=True, blk=ROW_BLK):
    m, k1 = x1.shape
    k2 = x2.shape[1]
    n = w1.shape[1]
    return pl.pallas_call(
        functools.partial(_mm2_kernel, act=act),
        grid=(m // blk,),
        in_specs=[
            pl.BlockSpec((blk, k1), lambda i: (i, 0)),
            pl.BlockSpec((blk, k2), lambda i: (i, 0)),
            pl.BlockSpec((k1, n), lambda i: (0, 0)),
            pl.BlockSpec((k2, n), lambda i: (0, 0)),
            pl.BlockSpec((1, n), lambda i: (0, 0)),
        ],
        out_specs=pl.BlockSpec((blk, n), lambda i: (i, 0)),
        out_shape=jax.ShapeDtypeStruct((m, n), jnp.float32),
    )(x1, x2, w1, w2, b.reshape(1, n))


def _gcu_ab_kernel(raw_ref, feat_ref, wp_ref, bp_ref, wt_ref, wb_ref, bc_ref,
                   *o_refs):
    pf = jnp.dot(raw_ref[...], wp_ref[...], preferred_element_type=jnp.float32)
    pf = jnp.maximum(pf + bp_ref[...], 0.0)
    y = jnp.dot(pf, wt_ref[...], preferred_element_type=jnp.float32)
    y = y + jnp.dot(feat_ref[...], wb_ref[...], preferred_element_type=jnp.float32)
    y = y + bc_ref[...]
    for t, o in enumerate(o_refs):
        o[...] = y[:, t * 128:(t + 1) * 128]


def _gcu_ab(raw, feat, wp, bp, wtop, wbot, bcat):
    m = raw.shape[0]
    kr = raw.shape[1]
    kf = feat.shape[1]
    outs = pl.pallas_call(
        _gcu_ab_kernel,
        grid=(m // ROW_BLK,),
        in_specs=[
            pl.BlockSpec((ROW_BLK, kr), lambda i: (i, 0)),
            pl.BlockSpec((ROW_BLK, kf), lambda i: (i, 0)),
            pl.BlockSpec((kr, 64), lambda i: (0, 0)),
            pl.BlockSpec((1, 64), lambda i: (0, 0)),
            pl.BlockSpec((64, 512), lambda i: (0, 0)),
            pl.BlockSpec((kf, 512), lambda i: (0, 0)),
            pl.BlockSpec((1, 512), lambda i: (0, 0)),
        ],
        out_specs=[pl.BlockSpec((ROW_BLK, 128), lambda i: (i, 0))] * 4,
        out_shape=[jax.ShapeDtypeStruct((m, 128), jnp.float32)] * 4,
    )(raw, feat, wp, bp.reshape(1, 64), wtop, wbot, bcat.reshape(1, 512))
    return outs


def _edge_mm_kernel(g_ref, w_ref, b_ref, o_ref):
    y = jnp.dot(g_ref[...], w_ref[...], preferred_element_type=jnp.float32)
    o_ref[...] = jnp.maximum(y + b_ref[...], 0.0)


def _edge_mm(g, w, b):
    e = g.shape[0]
    return pl.pallas_call(
        _edge_mm_kernel,
        grid=(e // EDGE_BLK,),
        in_specs=[
            pl.BlockSpec((EDGE_BLK, 128), lambda i: (i, 0)),
            pl.BlockSpec((128, 128), lambda i: (0, 0)),
            pl.BlockSpec((1, 128), lambda i: (0, 0)),
        ],
        out_specs=pl.BlockSpec((EDGE_BLK, 128), lambda i: (i, 0)),
        out_shape=jax.ShapeDtypeStruct((e, 128), jnp.float32),
    )(g, w, b.reshape(1, 128))


def _mlt2_segmax_kernel(x_ref, w_ref, b_ref, batch_ref, o_ref, acc_ref):
    i = pl.program_id(0)

    @pl.when(i == 0)
    def _():
        acc_ref[...] = jnp.full_like(acc_ref, NEG_BIG)

    y = jnp.dot(x_ref[...], w_ref[...], preferred_element_type=jnp.float32)
    y = jnp.maximum(y + b_ref[...], 0.0)
    bids = batch_ref[...]
    for g in range(NGRAPH):
        m = (bids == g)
        colmax = jnp.max(jnp.where(m, y, NEG_BIG), axis=0)
        acc_ref[g, :] = jnp.maximum(acc_ref[g, :], colmax)

    @pl.when(i == pl.num_programs(0) - 1)
    def _():
        a = acc_ref[...]
        o_ref[...] = jnp.where(a <= NEG_BIG, 0.0, a)


def _mlt2_segmax(x, w, b, batch):
    m, k = x.shape
    n = w.shape[1]
    return pl.pallas_call(
        _mlt2_segmax_kernel,
        grid=(m // ROW_BLK,),
        in_specs=[
            pl.BlockSpec((ROW_BLK, k), lambda i: (i, 0)),
            pl.BlockSpec((k, n), lambda i: (0, 0)),
            pl.BlockSpec((1, n), lambda i: (0, 0)),
            pl.BlockSpec((ROW_BLK, 1), lambda i: (i, 0)),
        ],
        out_specs=pl.BlockSpec((8, n), lambda i: (0, 0)),
        out_shape=jax.ShapeDtypeStruct((8, n), jnp.float32),
        scratch_shapes=[pltpu.VMEM((8, n), jnp.float32)],
    )(x, w, b.reshape(1, n), batch.reshape(m, 1))


def _cls_kernel(x3_ref, xg4_ref, batch_ref, wt_ref, wb_ref, b1_ref,
                w2_ref, b2_ref, w3_ref, b3_ref, o_ref):
    xgw = jnp.dot(xg4_ref[...], wb_ref[...], preferred_element_type=jnp.float32)
    bids = batch_ref[...]
    onehot = (bids == lax.broadcasted_iota(jnp.int32, (1, 8), 1)).astype(jnp.float32)
    gathered = jnp.dot(onehot, xgw, preferred_element_type=jnp.float32)
    h = jnp.dot(x3_ref[...], wt_ref[...], preferred_element_type=jnp.float32)
    h = jnp.maximum(h + gathered + b1_ref[...], 0.0)
    h = jnp.dot(h, w2_ref[...], preferred_element_type=jnp.float32)
    h = jnp.maximum(h + b2_ref[...], 0.0)
    y = jnp.dot(h, w3_ref[...], preferred_element_type=jnp.float32)
    o_ref[...] = y + b3_ref[...]


def _cls_head(x3, xg4, batch, wtop, wbot, b1, w2, b2, w3, b3):
    m = x3.shape[0]
    nout = w3.shape[1]
    return pl.pallas_call(
        _cls_kernel,
        grid=(m // ROW_BLK,),
        in_specs=[
            pl.BlockSpec((ROW_BLK, 256), lambda i: (i, 0)),
            pl.BlockSpec((8, 1024), lambda i: (0, 0)),
            pl.BlockSpec((ROW_BLK, 1), lambda i: (i, 0)),
            pl.BlockSpec((256, 1024), lambda i: (0, 0)),
            pl.BlockSpec((1024, 1024), lambda i: (0, 0)),
            pl.BlockSpec((1, 1024), lambda i: (0, 0)),
            pl.BlockSpec((1024, 512), lambda i: (0, 0)),
            pl.BlockSpec((1, 512), lambda i: (0, 0)),
            pl.BlockSpec((512, nout), lambda i: (0, 0)),
            pl.BlockSpec((1, nout), lambda i: (0, 0)),
        ],
        out_specs=pl.BlockSpec((ROW_BLK, nout), lambda i: (i, 0)),
        out_shape=jax.ShapeDtypeStruct((m, nout), jnp.float32),
    )(x3, xg4, batch.reshape(m, 1), wtop, wbot, b1.reshape(1, 1024),
      w2, b2.reshape(1, 512), w3, b3.reshape(1, nout))


_NC, _NS = 2, 16
_NW = _NC * _NS
_EPW = N_EPAD // _NW
_GCH = 896
_GH = _GCH // 2
_DN = 313
_DN1 = _DN + 1
_NP = _NW * _DN
_SCH = 4000
_GB = 32
_CBUF = _SCH + _GB + 32
_CB2 = _SCH + _GB
_CAP = 168192
_NCHK = N_EDGES // _SCH


def _sc_mesh():
    return plsc.VectorSubcoreMesh(core_axis_name="c", subcore_axis_name="s")


_GQ = _GCH // 4


def _sc_gather_add(a_tab, b_tab, dst, src):
    @functools.partial(
        pl.kernel, mesh=_sc_mesh(),
        compiler_params=pltpu.CompilerParams(needs_layout_passes=False),
        out_type=jax.ShapeDtypeStruct((N_EPAD, 128), jnp.float32),
        scratch_types=[
            pltpu.VMEM((_GCH,), jnp.int32),
            pltpu.VMEM((_GCH,), jnp.int32),
            pltpu.VMEM((_GQ, 128), jnp.float32),
            pltpu.VMEM((_GQ, 128), jnp.float32),
            pltpu.VMEM((_GQ, 128), jnp.float32),
            pltpu.VMEM((_GQ, 128), jnp.float32),
            pltpu.SemaphoreType.DMA,
            pltpu.SemaphoreType.DMA,
            pltpu.SemaphoreType.DMA,
            pltpu.SemaphoreType.DMA,
            pltpu.SemaphoreType.DMA,
            pltpu.SemaphoreType.DMA,
        ])
    def k(a_hbm, b_hbm, dst_hbm, src_hbm, g_hbm,
          di_v, si_v, a0, a1, b0, b1, ga0, ga1, gb0, gb1, w0, w1):
        wid = lax.axis_index("s") * _NC + lax.axis_index("c")
        base = wid * _EPW
        abufs, bbufs = (a0, a1), (b0, b1)
        gas, gbs, ws = (ga0, ga1), (gb0, gb1), (w0, w1)

        def body(c, carry):
            off = base + c * _GCH
            pltpu.sync_copy(dst_hbm.at[pl.ds(off, _GCH)], di_v)
            pltpu.sync_copy(src_hbm.at[pl.ds(off, _GCH)], si_v)

            def startg(q):
                p = q % 2
                pltpu.async_copy(a_hbm.at[di_v.at[pl.ds(q * _GQ, _GQ)]],
                                 abufs[p], gas[p])
                pltpu.async_copy(b_hbm.at[si_v.at[pl.ds(q * _GQ, _GQ)]],
                                 bbufs[p], gbs[p])

            startg(0)
            startg(1)
            for q in range(4):
                p = q % 2
                pltpu.make_async_copy(a_hbm.at[di_v.at[pl.ds(q * _GQ, _GQ)]],
                                      abufs[p], gas[p]).wait()
                pltpu.make_async_copy(b_hbm.at[si_v.at[pl.ds(q * _GQ, _GQ)]],
                                      bbufs[p], gbs[p]).wait()

                def fuse(r, cc):
                    for kk in range(8):
                        sl = pl.ds(kk * 16, 16)
                        av = abufs[p][r, sl]
                        bv = bbufs[p][r, sl]
                        abufs[p][r, sl] = jnp.maximum(av + bv, 0.0)
                    return cc

                lax.fori_loop(0, _GQ, fuse, 0)
                pltpu.async_copy(abufs[p],
                                 g_hbm.at[pl.ds(off + q * _GQ, _GQ)], ws[p])
                if q + 2 < 4:
                    pltpu.make_async_copy(
                        abufs[p], g_hbm.at[pl.ds(off + q * _GQ, _GQ)],
                        ws[p]).wait()
                    startg(q + 2)
            for q in (2, 3):
                p = q % 2
                pltpu.make_async_copy(
                    abufs[p], g_hbm.at[pl.ds(off + q * _GQ, _GQ)],
                    ws[p]).wait()
            return carry

        lax.fori_loop(0, _EPW // _GCH, body, 0)

    return k(a_tab, b_tab, dst, src)


def _sc_compact(dst):
    @functools.partial(
        pl.kernel, mesh=_sc_mesh(),
        compiler_params=pltpu.CompilerParams(needs_layout_passes=False),
        out_type=[jax.ShapeDtypeStruct((_NW * _CAP,), jnp.int32),
                  jax.ShapeDtypeStruct((_NW * _CAP,), jnp.int32),
                  jax.ShapeDtypeStruct((_NW * 16,), jnp.int32)],
        scratch_types=[
            pltpu.VMEM((_SCH,), jnp.int32),
            pltpu.VMEM((_CBUF,), jnp.int32),
            pltpu.VMEM((_CBUF,), jnp.int32),
            pltpu.VMEM((_CBUF,), jnp.int32),
            pltpu.VMEM((_CBUF,), jnp.int32),
            pltpu.VMEM((16,), jnp.int32),
            pltpu.SemaphoreType.DMA,
            pltpu.SemaphoreType.DMA,
            pltpu.SemaphoreType.DMA,
            pltpu.SemaphoreType.DMA,
        ])
    def k(dst_hbm, ce_hbm, cl_hbm, cnt_hbm,
          dbuf, ce0, ce1, cl0, cl1, cbuf, se0, se1, sl0, sl1):
        wid = lax.axis_index("s") * _NC + lax.axis_index("c")
        lo = wid * _DN
        base = wid * _CAP
        lane = lax.iota(jnp.int32, 16)
        dn_splat = jnp.full((16,), _DN, jnp.int32)
        zero_splat = jnp.full((16,), 0, jnp.int32)
        cebufs, clbufs = (ce0, ce1), (cl0, cl1)
        esems, lsems = (se0, se1), (sl0, sl1)

        def one_chunk(c, u, cg):
            pltpu.sync_copy(dst_hbm.at[pl.ds(c * _SCH, _SCH)], dbuf)

            def scan(j, ptr):
                vd = dbuf[pl.ds(j * 16, 16)]
                m = (vd >= lo) & (vd < lo + _DN)
                mi = jnp.where(m, 1, 0)
                pos = plsc.cumsum(mi)
                eid = c * _SCH + j * 16 + lane
                tgt = jnp.where(m, ptr + pos - 1, _CBUF - 1)
                plsc.store_scatter(cebufs[u], [tgt], eid)
                plsc.store_scatter(clbufs[u], [tgt], vd - lo)
                return ptr + jnp.max(pos)

            total = lax.fori_loop(0, _SCH // 16, scan, 0)
            cebufs[u][pl.ds(total, 16)] = zero_splat
            clbufs[u][pl.ds(total, 16)] = dn_splat
            pltpu.async_copy(cebufs[u].at[pl.ds(0, _SCH)],
                             ce_hbm.at[pl.ds(base + cg * 8, _SCH)], esems[u])
            pltpu.async_copy(clbufs[u].at[pl.ds(0, _SCH)],
                             cl_hbm.at[pl.ds(base + cg * 8, _SCH)], lsems[u])
            return cg + (total + 7) // 8

        def body(i, cg):
            for u in range(2):
                @pl.when(i > 0)
                def _():
                    pltpu.make_async_copy(
                        cebufs[u].at[pl.ds(0, _SCH)],
                        ce_hbm.at[pl.ds(base, _SCH)], esems[u]).wait()
                    pltpu.make_async_copy(
                        clbufs[u].at[pl.ds(0, _SCH)],
                        cl_hbm.at[pl.ds(base, _SCH)], lsems[u]).wait()
                cg = one_chunk(2 * i + u, u, cg)
            return cg

        cum = lax.fori_loop(0, _NCHK // 2, body, 0) * 8
        for u in range(2):
            pltpu.make_async_copy(cebufs[u].at[pl.ds(0, _SCH)],
                                  ce_hbm.at[pl.ds(base, _SCH)], esems[u]).wait()
            pltpu.make_async_copy(clbufs[u].at[pl.ds(0, _SCH)],
                                  cl_hbm.at[pl.ds(base, _SCH)], lsems[u]).wait()
        cbuf[pl.ds(0, 16)] = zero_splat + cum
        pltpu.sync_copy(cbuf, cnt_hbm.at[pl.ds(wid * 16, 16)])

    return k(dst)


def _sc_segmax(h, comp):
    ce_all, cl_all, cnts = comp
    zeros_f = jnp.zeros((_DN1 * 128,), jnp.float32)

    @functools.partial(
        pl.kernel, mesh=_sc_mesh(),
        compiler_params=pltpu.CompilerParams(needs_layout_passes=False),
        out_type=jax.ShapeDtypeStruct((_NP * 128,), jnp.float32),
        scratch_types=[
            pltpu.VMEM((_DN1 * 128,), jnp.float32),
            pltpu.VMEM((_CB2,), jnp.int32),
            pltpu.VMEM((_CB2,), jnp.int32),
            pltpu.VMEM((16,), jnp.int32),
            pltpu.VMEM((_GB, 128), jnp.float32),
            pltpu.VMEM((_GB, 128), jnp.float32),
            pltpu.SemaphoreType.DMA,
            pltpu.SemaphoreType.DMA,
        ])
    def k(h_hbm, ce_hbm, cl_hbm, cnt_hbm, z_hbm, out_hbm,
          out_l, cebuf, clbuf, cbuf, hbuf0, hbuf1, sem0, sem1):
        wid = lax.axis_index("s") * _NC + lax.axis_index("c")
        lo = wid * _DN
        base = wid * _CAP
        lane = lax.iota(jnp.int32, 16)

        pltpu.sync_copy(z_hbm, out_l)
        pltpu.sync_copy(cnt_hbm.at[pl.ds(wid * 16, 16)], cbuf)
        cnt = jnp.max(cbuf[pl.ds(0, 16)])
        nch = (cnt + _SCH - 1) // _SCH

        def chunk(cix, c0):
            pltpu.sync_copy(ce_hbm.at[pl.ds(base + cix * _SCH, _SCH)],
                            cebuf.at[pl.ds(0, _SCH)])
            pltpu.sync_copy(cl_hbm.at[pl.ds(base + cix * _SCH, _SCH)],
                            clbuf.at[pl.ds(0, _SCH)])
            valid = jnp.minimum(cnt - cix * _SCH, _SCH)

            def mask(j, cc):
                idx = j * 16 + lane
                mm = idx < valid
                ce = cebuf[pl.ds(j * 16, 16)]
                cl = clbuf[pl.ds(j * 16, 16)]
                cebuf[pl.ds(j * 16, 16)] = jnp.where(mm, ce, 0)
                clbuf[pl.ds(j * 16, 16)] = jnp.where(mm, cl, _DN)
                return cc

            lax.fori_loop(0, _CB2 // 16, mask, 0)
            nsub = (valid + _GB - 1) // _GB

            def start(s, buf, sem):
                pltpu.async_copy(h_hbm.at[cebuf.at[pl.ds(s * _GB, _GB)]],
                                 buf, sem)

            def rmw(s, buf, sem):
                pltpu.make_async_copy(h_hbm.at[cebuf.at[pl.ds(s * _GB, _GB)]],
                                      buf, sem).wait()
                for gq in range(_GB // 16):
                    dvec = clbuf[pl.ds(s * _GB + gq * 16, 16)]
                    for i in range(16):
                        dd = jnp.max(jnp.where(lane == i, dvec, 0))
                        rbase = dd * 128
                        for kk in range(8):
                            sl = pl.ds(rbase + kk * 16, 16)
                            hv = buf[gq * 16 + i, pl.ds(kk * 16, 16)]
                            out_l[sl] = jnp.maximum(out_l[sl], hv)

            @pl.when(nsub > 0)
            def _():
                start(0, hbuf0, sem0)

            def sub2(i, c1):
                s0 = 2 * i

                @pl.when(s0 + 1 < nsub)
                def _():
                    start(s0 + 1, hbuf1, sem1)

                rmw(s0, hbuf0, sem0)

                @pl.when(s0 + 2 < nsub)
                def _():
                    start(s0 + 2, hbuf0, sem0)

                @pl.when(s0 + 1 < nsub)
                def _():
                    rmw(s0 + 1, hbuf1, sem1)

                return c1

            lax.fori_loop(0, (nsub + 1) // 2, sub2, 0)
            return c0

        lax.fori_loop(0, nch, chunk, 0)
        pltpu.sync_copy(out_l.at[pl.ds(0, _DN * 128)],
                        out_hbm.at[pl.ds(lo * 128, _DN * 128)])

    out = k(h, ce_all, cl_all, cnts, zeros_f)
    return out.reshape(_NP, 128)[:N_NODES]


def _edge_conv(a_tab, b_tab, w2, b2, dst, src, comp):
    g = _sc_gather_add(a_tab, b_tab, dst, src)
    h = _edge_mm(g, w2, b2)
    return _sc_segmax(h, comp)


def _split_edge_params(gcu_p, xdim):
    out = {}
    for et in ('tpl', 'geo'):
        (w1, b1), (w2, b2) = gcu_p[et]
        w1a, w1b = w1[:xdim], w1[xdim:]
        out[et] = (w1a - w1b, w1b, b1, w2, b2)
    return out


def _gcu_layer(gcu_p, raw, feat, dst_t, src_t, dst_g, src_g, comp_t, comp_g):
    xdim = 64 + feat.shape[1]
    ep = _split_edge_params(gcu_p, xdim)
    (wp, bp), = gcu_p['pos_mlp']
    wtop = jnp.concatenate([ep['tpl'][0][:64], ep['tpl'][1][:64],
                            ep['geo'][0][:64], ep['geo'][1][:64]], axis=1)
    wbot = jnp.concatenate([ep['tpl'][0][64:], ep['tpl'][1][64:],
                            ep['geo'][0][64:], ep['geo'][1][64:]], axis=1)
    zeros = jnp.zeros_like(ep['tpl'][2])
    bcat = jnp.concatenate([ep['tpl'][2], zeros, ep['geo'][2], zeros])
    a_t, b_t, a_g, b_g = _gcu_ab(raw, feat, wp, bp, wtop, wbot, bcat)
    xt = _edge_conv(a_t, b_t, ep['tpl'][3], ep['tpl'][4], dst_t, src_t, comp_t)
    xg = _edge_conv(a_g, b_g, ep['geo'][3], ep['geo'][4], dst_g, src_g, comp_g)
    (wm, bm), = gcu_p['mlp']
    return _mm2(xt, xg, wm[:128], wm[128:], bm)


def kernel(pos, skin_input, motion, tpl_edge_index, geo_edge_index, batch, params):
    raw = jnp.concatenate([pos, skin_input[:, :8 * NB]], axis=1)
    pad = (jnp.arange(N_EPAD - N_EDGES, dtype=jnp.int32) * 13) % N_NODES
    def _p(v):
        return jnp.concatenate([v, pad])
    src_t, dst_t = _p(tpl_edge_index[0]), _p(tpl_edge_index[1])
    src_g, dst_g = _p(geo_edge_index[0]), _p(geo_edge_index[1])
    comp_t = _sc_compact(dst_t)
    comp_g = _sc_compact(dst_g)

    x1 = _gcu_layer(params['gcu1'], raw, motion, dst_t, src_t, dst_g, src_g,
                    comp_t, comp_g)

    (wl1, bl1), (wl2, bl2) = params['mlt2']
    h = _mm(x1, wl1, bl1)
    xg4 = _mlt2_segmax(h, wl2, bl2, batch)

    x2 = _gcu_layer(params['gcu2'], raw, x1, dst_t, src_t, dst_g, src_g,
                    comp_t, comp_g)
    x3 = _gcu_layer(params['gcu3'], raw, x2, dst_t, src_t, dst_g, src_g,
                    comp_t, comp_g)

    (wc1, bc1), (wc2, bc2) = params['cls_mlp']
    wo, bo = params['cls_out']
    return _cls_head(x3, xg4, batch, wc1[:256], wc1[256:], bc1,
                     wc2, bc2, wo, bo)


# speedup vs baseline: 1.4224x; 1.0747x over previous
import functools

import jax
import jax.numpy as jnp
from jax import lax
from jax.experimental import pallas as pl
from jax.experimental.pallas import tpu as pltpu
from jax.experimental.pallas import tpu_sc as plsc

N_NODES = 10000
N_EDGES = 160000
N_EPAD = 172032
NB = 5
NGRAPH = 4
ROW_BLK = 2000
EDGE_BLK = 4096
NEG_BIG = -3.0e38


def _mm_kernel(x_ref, w_ref, b_ref, o_ref, *, act):
    y = jnp.dot(x_ref[...], w_ref[...], preferred_element_type=jnp.float32)
    y = y + b_ref[...]
    if act:
        y = jnp.maximum(y, 0.0)
    o_ref[...] = y


def _mm(x, w, b, act=True, blk=ROW_BLK):
    m, k = x.shape
    n = w.shape[1]
    return pl.pallas_call(
        functools.partial(_mm_kernel, act=act),
        grid=(m // blk,),
        in_specs=[
            pl.BlockSpec((blk, k), lambda i: (i, 0)),
            pl.BlockSpec((k, n), lambda i: (0, 0)),
            pl.BlockSpec((1, n), lambda i: (0, 0)),
        ],
        out_specs=pl.BlockSpec((blk, n), lambda i: (i, 0)),
        out_shape=jax.ShapeDtypeStruct((m, n), jnp.float32),
    )(x, w, b.reshape(1, n))


def _mm2_kernel(x1_ref, x2_ref, w1_ref, w2_ref, b_ref, o_ref, *, act):
    y = jnp.dot(x1_ref[...], w1_ref[...], preferred_element_type=jnp.float32)
    y = y + jnp.dot(x2_ref[...], w2_ref[...], preferred_element_type=jnp.float32)
    y = y + b_ref[...]
    if act:
        y = jnp.maximum(y, 0.0)
    o_ref[...] = y


def _mm2(x1, x2, w1, w2, b, act=True, blk=ROW_BLK):
    m, k1 = x1.shape
    k2 = x2.shape[1]
    n = w1.shape[1]
    return pl.pallas_call(
        functools.partial(_mm2_kernel, act=act),
        grid=(m // blk,),
        in_specs=[
            pl.BlockSpec((blk, k1), lambda i: (i, 0)),
            pl.BlockSpec((blk, k2), lambda i: (i, 0)),
            pl.BlockSpec((k1, n), lambda i: (0, 0)),
            pl.BlockSpec((k2, n), lambda i: (0, 0)),
            pl.BlockSpec((1, n), lambda i: (0, 0)),
        ],
        out_specs=pl.BlockSpec((blk, n), lambda i: (i, 0)),
        out_shape=jax.ShapeDtypeStruct((m, n), jnp.float32),
    )(x1, x2, w1, w2, b.reshape(1, n))


def _gcu_ab_kernel(raw_ref, feat_ref, wp_ref, bp_ref, wt_ref, wb_ref, bc_ref,
                   *o_refs):
    pf = jnp.dot(raw_ref[...], wp_ref[...], preferred_element_type=jnp.float32)
    pf = jnp.maximum(pf + bp_ref[...], 0.0)
    y = jnp.dot(pf, wt_ref[...], preferred_element_type=jnp.float32)
    y = y + jnp.dot(feat_ref[...], wb_ref[...], preferred_element_type=jnp.float32)
    y = y + bc_ref[...]
    for t, o in enumerate(o_refs):
        o[...] = y[:, t * 128:(t + 1) * 128]


def _gcu_ab(raw, feat, wp, bp, wtop, wbot, bcat):
    m = raw.shape[0]
    kr = raw.shape[1]
    kf = feat.shape[1]
    outs = pl.pallas_call(
        _gcu_ab_kernel,
        grid=(m // ROW_BLK,),
        in_specs=[
            pl.BlockSpec((ROW_BLK, kr), lambda i: (i, 0)),
            pl.BlockSpec((ROW_BLK, kf), lambda i: (i, 0)),
            pl.BlockSpec((kr, 64), lambda i: (0, 0)),
            pl.BlockSpec((1, 64), lambda i: (0, 0)),
            pl.BlockSpec((64, 512), lambda i: (0, 0)),
            pl.BlockSpec((kf, 512), lambda i: (0, 0)),
            pl.BlockSpec((1, 512), lambda i: (0, 0)),
        ],
        out_specs=[pl.BlockSpec((ROW_BLK, 128), lambda i: (i, 0))] * 4,
        out_shape=[jax.ShapeDtypeStruct((m, 128), jnp.float32)] * 4,
    )(raw, feat, wp, bp.reshape(1, 64), wtop, wbot, bcat.reshape(1, 512))
    return outs


def _edge_mm_kernel(g_ref, w_ref, b_ref, o_ref):
    y = jnp.dot(g_ref[...], w_ref[...], preferred_element_type=jnp.float32)
    o_ref[...] = jnp.maximum(y + b_ref[...], 0.0)


def _edge_mm(g, w, b):
    e = g.shape[0]
    return pl.pallas_call(
        _edge_mm_kernel,
        grid=(e // EDGE_BLK,),
        in_specs=[
            pl.BlockSpec((EDGE_BLK, 128), lambda i: (i, 0)),
            pl.BlockSpec((128, 128), lambda i: (0, 0)),
            pl.BlockSpec((1, 128), lambda i: (0, 0)),
        ],
        out_specs=pl.BlockSpec((EDGE_BLK, 128), lambda i: (i, 0)),
        out_shape=jax.ShapeDtypeStruct((e, 128), jnp.float32),
    )(g, w, b.reshape(1, 128))


def _mlt2_segmax_kernel(x_ref, w_ref, b_ref, batch_ref, o_ref, acc_ref):
    i = pl.program_id(0)

    @pl.when(i == 0)
    def _():
        acc_ref[...] = jnp.full_like(acc_ref, NEG_BIG)

    y = jnp.dot(x_ref[...], w_ref[...], preferred_element_type=jnp.float32)
    y = jnp.maximum(y + b_ref[...], 0.0)
    bids = batch_ref[...]
    for g in range(NGRAPH):
        m = (bids == g)
        colmax = jnp.max(jnp.where(m, y, NEG_BIG), axis=0)
        acc_ref[g, :] = jnp.maximum(acc_ref[g, :], colmax)

    @pl.when(i == pl.num_programs(0) - 1)
    def _():
        a = acc_ref[...]
        o_ref[...] = jnp.where(a <= NEG_BIG, 0.0, a)


def _mlt2_segmax(x, w, b, batch):
    m, k = x.shape
    n = w.shape[1]
    return pl.pallas_call(
        _mlt2_segmax_kernel,
        grid=(m // ROW_BLK,),
        in_specs=[
            pl.BlockSpec((ROW_BLK, k), lambda i: (i, 0)),
            pl.BlockSpec((k, n), lambda i: (0, 0)),
            pl.BlockSpec((1, n), lambda i: (0, 0)),
            pl.BlockSpec((ROW_BLK, 1), lambda i: (i, 0)),
        ],
        out_specs=pl.BlockSpec((8, n), lambda i: (0, 0)),
        out_shape=jax.ShapeDtypeStruct((8, n), jnp.float32),
        scratch_shapes=[pltpu.VMEM((8, n), jnp.float32)],
    )(x, w, b.reshape(1, n), batch.reshape(m, 1))


def _cls_kernel(x3_ref, xg4_ref, batch_ref, wt_ref, wb_ref, b1_ref,
                w2_ref, b2_ref, w3_ref, b3_ref, o_ref):
    xgw = jnp.dot(xg4_ref[...], wb_ref[...], preferred_element_type=jnp.float32)
    bids = batch_ref[...]
    onehot = (bids == lax.broadcasted_iota(jnp.int32, (1, 8), 1)).astype(jnp.float32)
    gathered = jnp.dot(onehot, xgw, preferred_element_type=jnp.float32)
    h = jnp.dot(x3_ref[...], wt_ref[...], preferred_element_type=jnp.float32)
    h = jnp.maximum(h + gathered + b1_ref[...], 0.0)
    h = jnp.dot(h, w2_ref[...], preferred_element_type=jnp.float32)
    h = jnp.maximum(h + b2_ref[...], 0.0)
    y = jnp.dot(h, w3_ref[...], preferred_element_type=jnp.float32)
    o_ref[...] = y + b3_ref[...]


def _cls_head(x3, xg4, batch, wtop, wbot, b1, w2, b2, w3, b3):
    m = x3.shape[0]
    nout = w3.shape[1]
    return pl.pallas_call(
        _cls_kernel,
        grid=(m // ROW_BLK,),
        in_specs=[
            pl.BlockSpec((ROW_BLK, 256), lambda i: (i, 0)),
            pl.BlockSpec((8, 1024), lambda i: (0, 0)),
            pl.BlockSpec((ROW_BLK, 1), lambda i: (i, 0)),
            pl.BlockSpec((256, 1024), lambda i: (0, 0)),
            pl.BlockSpec((1024, 1024), lambda i: (0, 0)),
            pl.BlockSpec((1, 1024), lambda i: (0, 0)),
            pl.BlockSpec((1024, 512), lambda i: (0, 0)),
            pl.BlockSpec((1, 512), lambda i: (0, 0)),
            pl.BlockSpec((512, nout), lambda i: (0, 0)),
            pl.BlockSpec((1, nout), lambda i: (0, 0)),
        ],
        out_specs=pl.BlockSpec((ROW_BLK, nout), lambda i: (i, 0)),
        out_shape=jax.ShapeDtypeStruct((m, nout), jnp.float32),
    )(x3, xg4, batch.reshape(m, 1), wtop, wbot, b1.reshape(1, 1024),
      w2, b2.reshape(1, 512), w3, b3.reshape(1, nout))


_NC, _NS = 2, 16
_NW = _NC * _NS
_EPW = N_EPAD // _NW
_GCH = 896
_GH = _GCH // 2
_DN = 313
_DN1 = _DN + 1
_NP = _NW * _DN
_SCH = 4000
_GB = 16
_CBUF = _SCH + _GB + 32
_CB2 = _SCH + _GB
_CAP = 168192
_NCHK = N_EDGES // _SCH


def _sc_mesh():
    return plsc.VectorSubcoreMesh(core_axis_name="c", subcore_axis_name="s")


_GQ = _GCH // 4


def _sc_gather_add(a_tab, b_tab, dst, src):
    @functools.partial(
        pl.kernel, mesh=_sc_mesh(),
        compiler_params=pltpu.CompilerParams(needs_layout_passes=False),
        out_type=jax.ShapeDtypeStruct((N_EPAD, 128), jnp.float32),
        scratch_types=[
            pltpu.VMEM((_GCH,), jnp.int32),
            pltpu.VMEM((_GCH,), jnp.int32),
            pltpu.VMEM((_GQ, 128), jnp.float32),
            pltpu.VMEM((_GQ, 128), jnp.float32),
            pltpu.VMEM((_GQ, 128), jnp.float32),
            pltpu.VMEM((_GQ, 128), jnp.float32),
            pltpu.SemaphoreType.DMA,
            pltpu.SemaphoreType.DMA,
            pltpu.SemaphoreType.DMA,
            pltpu.SemaphoreType.DMA,
            pltpu.SemaphoreType.DMA,
            pltpu.SemaphoreType.DMA,
        ])
    def k(a_hbm, b_hbm, dst_hbm, src_hbm, g_hbm,
          di_v, si_v, a0, a1, b0, b1, ga0, ga1, gb0, gb1, w0, w1):
        wid = lax.axis_index("s") * _NC + lax.axis_index("c")
        base = wid * _EPW
        abufs, bbufs = (a0, a1), (b0, b1)
        gas, gbs, ws = (ga0, ga1), (gb0, gb1), (w0, w1)

        def body(c, carry):
            off = base + c * _GCH
            pltpu.sync_copy(dst_hbm.at[pl.ds(off, _GCH)], di_v)
            pltpu.sync_copy(src_hbm.at[pl.ds(off, _GCH)], si_v)

            def startg(q):
                p = q % 2
                pltpu.async_copy(a_hbm.at[di_v.at[pl.ds(q * _GQ, _GQ)]],
                                 abufs[p], gas[p])
                pltpu.async_copy(b_hbm.at[si_v.at[pl.ds(q * _GQ, _GQ)]],
                                 bbufs[p], gbs[p])

            startg(0)
            startg(1)
            for q in range(4):
                p = q % 2
                pltpu.make_async_copy(a_hbm.at[di_v.at[pl.ds(q * _GQ, _GQ)]],
                                      abufs[p], gas[p]).wait()
                pltpu.make_async_copy(b_hbm.at[si_v.at[pl.ds(q * _GQ, _GQ)]],
                                      bbufs[p], gbs[p]).wait()

                def fuse(r, cc):
                    for kk in range(8):
                        sl = pl.ds(kk * 16, 16)
                        av = abufs[p][r, sl]
                        bv = bbufs[p][r, sl]
                        abufs[p][r, sl] = jnp.maximum(av + bv, 0.0)
                    return cc

                lax.fori_loop(0, _GQ, fuse, 0)
                pltpu.async_copy(abufs[p],
                                 g_hbm.at[pl.ds(off + q * _GQ, _GQ)], ws[p])
                if q + 2 < 4:
                    pltpu.make_async_copy(
                        abufs[p], g_hbm.at[pl.ds(off + q * _GQ, _GQ)],
                        ws[p]).wait()
                    startg(q + 2)
            for q in (2, 3):
                p = q % 2
                pltpu.make_async_copy(
                    abufs[p], g_hbm.at[pl.ds(off + q * _GQ, _GQ)],
                    ws[p]).wait()
            return carry

        lax.fori_loop(0, _EPW // _GCH, body, 0)

    return k(a_tab, b_tab, dst, src)


def _sc_compact(dst):
    @functools.partial(
        pl.kernel, mesh=_sc_mesh(),
        compiler_params=pltpu.CompilerParams(needs_layout_passes=False),
        out_type=[jax.ShapeDtypeStruct((_NW * _CAP,), jnp.int32),
                  jax.ShapeDtypeStruct((_NW * _CAP,), jnp.int32),
                  jax.ShapeDtypeStruct((_NW * 16,), jnp.int32)],
        scratch_types=[
            pltpu.VMEM((_SCH,), jnp.int32),
            pltpu.VMEM((_CBUF,), jnp.int32),
            pltpu.VMEM((_CBUF,), jnp.int32),
            pltpu.VMEM((_CBUF,), jnp.int32),
            pltpu.VMEM((_CBUF,), jnp.int32),
            pltpu.VMEM((16,), jnp.int32),
            pltpu.SemaphoreType.DMA,
            pltpu.SemaphoreType.DMA,
            pltpu.SemaphoreType.DMA,
            pltpu.SemaphoreType.DMA,
        ])
    def k(dst_hbm, ce_hbm, cl_hbm, cnt_hbm,
          dbuf, ce0, ce1, cl0, cl1, cbuf, se0, se1, sl0, sl1):
        wid = lax.axis_index("s") * _NC + lax.axis_index("c")
        lo = wid * _DN
        base = wid * _CAP
        lane = lax.iota(jnp.int32, 16)
        dn_splat = jnp.full((16,), _DN, jnp.int32)
        zero_splat = jnp.full((16,), 0, jnp.int32)
        cebufs, clbufs = (ce0, ce1), (cl0, cl1)
        esems, lsems = (se0, se1), (sl0, sl1)

        def one_chunk(c, u, cg):
            pltpu.sync_copy(dst_hbm.at[pl.ds(c * _SCH, _SCH)], dbuf)

            def scan(j, ptr):
                vd = dbuf[pl.ds(j * 16, 16)]
                m = (vd >= lo) & (vd < lo + _DN)
                mi = jnp.where(m, 1, 0)
                pos = plsc.cumsum(mi)
                eid = c * _SCH + j * 16 + lane
                tgt = jnp.where(m, ptr + pos - 1, _CBUF - 1)
                plsc.store_scatter(cebufs[u], [tgt], eid)
                plsc.store_scatter(clbufs[u], [tgt], vd - lo)
                return ptr + jnp.max(pos)

            total = lax.fori_loop(0, _SCH // 16, scan, 0)
            cebufs[u][pl.ds(total, 16)] = zero_splat
            clbufs[u][pl.ds(total, 16)] = dn_splat
            pltpu.async_copy(cebufs[u].at[pl.ds(0, _SCH)],
                             ce_hbm.at[pl.ds(base + cg * 8, _SCH)], esems[u])
            pltpu.async_copy(clbufs[u].at[pl.ds(0, _SCH)],
                             cl_hbm.at[pl.ds(base + cg * 8, _SCH)], lsems[u])
            return cg + (total + 7) // 8

        def body(i, cg):
            for u in range(2):
                @pl.when(i > 0)
                def _():
                    pltpu.make_async_copy(
                        cebufs[u].at[pl.ds(0, _SCH)],
                        ce_hbm.at[pl.ds(base, _SCH)], esems[u]).wait()
                    pltpu.make_async_copy(
                        clbufs[u].at[pl.ds(0, _SCH)],
                        cl_hbm.at[pl.ds(base, _SCH)], lsems[u]).wait()
                cg = one_chunk(2 * i + u, u, cg)
            return cg

        cum = lax.fori_loop(0, _NCHK // 2, body, 0) * 8
        for u in range(2):
            pltpu.make_async_copy(cebufs[u].at[pl.ds(0, _SCH)],
                                  ce_hbm.at[pl.ds(base, _SCH)], esems[u]).wait()
            pltpu.make_async_copy(clbufs[u].at[pl.ds(0, _SCH)],
                                  cl_hbm.at[pl.ds(base, _SCH)], lsems[u]).wait()
        cbuf[pl.ds(0, 16)] = zero_splat + cum
        pltpu.sync_copy(cbuf, cnt_hbm.at[pl.ds(wid * 16, 16)])

    return k(dst)


def _sc_segmax(h, comp):
    ce_all, cl_all, cnts = comp
    zeros_f = jnp.zeros((_DN1 * 128,), jnp.float32)

    @functools.partial(
        pl.kernel, mesh=_sc_mesh(),
        compiler_params=pltpu.CompilerParams(needs_layout_passes=False),
        out_type=jax.ShapeDtypeStruct((_NP * 128,), jnp.float32),
        scratch_types=[
            pltpu.VMEM((_DN1 * 128,), jnp.float32),
            pltpu.VMEM((_CB2,), jnp.int32),
            pltpu.VMEM((_CB2,), jnp.int32),
            pltpu.VMEM((16,), jnp.int32),
            pltpu.VMEM((_GB, 128), jnp.float32),
            pltpu.VMEM((_GB, 128), jnp.float32),
            pltpu.SemaphoreType.DMA,
            pltpu.SemaphoreType.DMA,
        ])
    def k(h_hbm, ce_hbm, cl_hbm, cnt_hbm, z_hbm, out_hbm,
          out_l, cebuf, clbuf, cbuf, hbuf0, hbuf1, sem0, sem1):
        wid = lax.axis_index("s") * _NC + lax.axis_index("c")
        lo = wid * _DN
        base = wid * _CAP
        lane = lax.iota(jnp.int32, 16)

        pltpu.sync_copy(z_hbm, out_l)
        pltpu.sync_copy(cnt_hbm.at[pl.ds(wid * 16, 16)], cbuf)
        cnt = jnp.max(cbuf[pl.ds(0, 16)])
        nch = (cnt + _SCH - 1) // _SCH

        def chunk(cix, c0):
            pltpu.sync_copy(ce_hbm.at[pl.ds(base + cix * _SCH, _SCH)],
                            cebuf.at[pl.ds(0, _SCH)])
            pltpu.sync_copy(cl_hbm.at[pl.ds(base + cix * _SCH, _SCH)],
                            clbuf.at[pl.ds(0, _SCH)])
            valid = jnp.minimum(cnt - cix * _SCH, _SCH)

            def mask(j, cc):
                idx = j * 16 + lane
                mm = idx < valid
                ce = cebuf[pl.ds(j * 16, 16)]
                cl = clbuf[pl.ds(j * 16, 16)]
                cebuf[pl.ds(j * 16, 16)] = jnp.where(mm, ce, 0)
                clbuf[pl.ds(j * 16, 16)] = jnp.where(mm, cl, _DN)
                return cc

            lax.fori_loop(0, _CB2 // 16, mask, 0)
            nsub = (valid + _GB - 1) // _GB

            def start(s, buf, sem):
                pltpu.async_copy(h_hbm.at[cebuf.at[pl.ds(s * _GB, _GB)]],
                                 buf, sem)

            def rmw(s, buf, sem):
                pltpu.make_async_copy(h_hbm.at[cebuf.at[pl.ds(s * _GB, _GB)]],
                                      buf, sem).wait()
                for gq in range(_GB // 16):
                    dvec = clbuf[pl.ds(s * _GB + gq * 16, 16)]
                    for i in range(16):
                        dd = jnp.max(jnp.where(lane == i, dvec, 0))
                        rbase = dd * 128
                        for kk in range(8):
                            sl = pl.ds(rbase + kk * 16, 16)
                            hv = buf[gq * 16 + i, pl.ds(kk * 16, 16)]
                            out_l[sl] = jnp.maximum(out_l[sl], hv)

            @pl.when(nsub > 0)
            def _():
                start(0, hbuf0, sem0)

            def sub2(i, c1):
                s0 = 2 * i

                @pl.when(s0 + 1 < nsub)
                def _():
                    start(s0 + 1, hbuf1, sem1)

                rmw(s0, hbuf0, sem0)

                @pl.when(s0 + 2 < nsub)
                def _():
                    start(s0 + 2, hbuf0, sem0)

                @pl.when(s0 + 1 < nsub)
                def _():
                    rmw(s0 + 1, hbuf1, sem1)

                return c1

            lax.fori_loop(0, (nsub + 1) // 2, sub2, 0)
            return c0

        lax.fori_loop(0, nch, chunk, 0)
        pltpu.sync_copy(out_l.at[pl.ds(0, _DN * 128)],
                        out_hbm.at[pl.ds(lo * 128, _DN * 128)])

    out = k(h, ce_all, cl_all, cnts, zeros_f)
    return out.reshape(_NP, 128)[:N_NODES]


def _edge_conv(a_tab, b_tab, w2, b2, dst, src, comp):
    g = _sc_gather_add(a_tab, b_tab, dst, src)
    h = _edge_mm(g, w2, b2)
    return _sc_segmax(h, comp)


def _split_edge_params(gcu_p, xdim):
    out = {}
    for et in ('tpl', 'geo'):
        (w1, b1), (w2, b2) = gcu_p[et]
        w1a, w1b = w1[:xdim], w1[xdim:]
        out[et] = (w1a - w1b, w1b, b1, w2, b2)
    return out


def _gcu_layer(gcu_p, raw, feat, dst_t, src_t, dst_g, src_g, comp_t, comp_g):
    xdim = 64 + feat.shape[1]
    ep = _split_edge_params(gcu_p, xdim)
    (wp, bp), = gcu_p['pos_mlp']
    wtop = jnp.concatenate([ep['tpl'][0][:64], ep['tpl'][1][:64],
                            ep['geo'][0][:64], ep['geo'][1][:64]], axis=1)
    wbot = jnp.concatenate([ep['tpl'][0][64:], ep['tpl'][1][64:],
                            ep['geo'][0][64:], ep['geo'][1][64:]], axis=1)
    zeros = jnp.zeros_like(ep['tpl'][2])
    bcat = jnp.concatenate([ep['tpl'][2], zeros, ep['geo'][2], zeros])
    a_t, b_t, a_g, b_g = _gcu_ab(raw, feat, wp, bp, wtop, wbot, bcat)
    xt = _edge_conv(a_t, b_t, ep['tpl'][3], ep['tpl'][4], dst_t, src_t, comp_t)
    xg = _edge_conv(a_g, b_g, ep['geo'][3], ep['geo'][4], dst_g, src_g, comp_g)
    (wm, bm), = gcu_p['mlp']
    return _mm2(xt, xg, wm[:128], wm[128:], bm)


def kernel(pos, skin_input, motion, tpl_edge_index, geo_edge_index, batch, params):
    raw = jnp.concatenate([pos, skin_input[:, :8 * NB]], axis=1)
    pad = (jnp.arange(N_EPAD - N_EDGES, dtype=jnp.int32) * 13) % N_NODES
    def _p(v):
        return jnp.concatenate([v, pad])
    src_t, dst_t = _p(tpl_edge_index[0]), _p(tpl_edge_index[1])
    src_g, dst_g = _p(geo_edge_index[0]), _p(geo_edge_index[1])
    comp_t = _sc_compact(dst_t)
    comp_g = _sc_compact(dst_g)

    x1 = _gcu_layer(params['gcu1'], raw, motion, dst_t, src_t, dst_g, src_g,
                    comp_t, comp_g)

    (wl1, bl1), (wl2, bl2) = params['mlt2']
    h = _mm(x1, wl1, bl1)
    xg4 = _mlt2_segmax(h, wl2, bl2, batch)

    x2 = _gcu_layer(params['gcu2'], raw, x1, dst_t, src_t, dst_g, src_g,
                    comp_t, comp_g)
    x3 = _gcu_layer(params['gcu3'], raw, x2, dst_t, src_t, dst_g, src_g,
                    comp_t, comp_g)

    (wc1, bc1), (wc2, bc2) = params['cls_mlp']
    wo, bo = params['cls_out']
    return _cls_head(x3, xg4, batch, wc1[:256], wc1[256:], bc1,
                     wc2, bc2, wo, bo)


# speedup vs baseline: 1.4435x; 1.0148x over previous
import functools

import jax
import jax.numpy as jnp
from jax import lax
from jax.experimental import pallas as pl
from jax.experimental.pallas import tpu as pltpu
from jax.experimental.pallas import tpu_sc as plsc

N_NODES = 10000
N_EDGES = 160000
N_EPAD = 172032
NB = 5
NGRAPH = 4
ROW_BLK = 2000
EDGE_BLK = 4096
NEG_BIG = -3.0e38


def _mm_kernel(x_ref, w_ref, b_ref, o_ref, *, act):
    y = jnp.dot(x_ref[...], w_ref[...], preferred_element_type=jnp.float32)
    y = y + b_ref[...]
    if act:
        y = jnp.maximum(y, 0.0)
    o_ref[...] = y


def _mm(x, w, b, act=True, blk=ROW_BLK):
    m, k = x.shape
    n = w.shape[1]
    return pl.pallas_call(
        functools.partial(_mm_kernel, act=act),
        grid=(m // blk,),
        in_specs=[
            pl.BlockSpec((blk, k), lambda i: (i, 0)),
            pl.BlockSpec((k, n), lambda i: (0, 0)),
            pl.BlockSpec((1, n), lambda i: (0, 0)),
        ],
        out_specs=pl.BlockSpec((blk, n), lambda i: (i, 0)),
        out_shape=jax.ShapeDtypeStruct((m, n), jnp.float32),
    )(x, w, b.reshape(1, n))


def _mm2_kernel(x1_ref, x2_ref, w1_ref, w2_ref, b_ref, o_ref, *, act):
    y = jnp.dot(x1_ref[...], w1_ref[...], preferred_element_type=jnp.float32)
    y = y + jnp.dot(x2_ref[...], w2_ref[...], preferred_element_type=jnp.float32)
    y = y + b_ref[...]
    if act:
        y = jnp.maximum(y, 0.0)
    o_ref[...] = y


def _mm2(x1, x2, w1, w2, b, act=True, blk=ROW_BLK):
    m, k1 = x1.shape
    k2 = x2.shape[1]
    n = w1.shape[1]
    return pl.pallas_call(
        functools.partial(_mm2_kernel, act=act),
        grid=(m // blk,),
        in_specs=[
            pl.BlockSpec((blk, k1), lambda i: (i, 0)),
            pl.BlockSpec((blk, k2), lambda i: (i, 0)),
            pl.BlockSpec((k1, n), lambda i: (0, 0)),
            pl.BlockSpec((k2, n), lambda i: (0, 0)),
            pl.BlockSpec((1, n), lambda i: (0, 0)),
        ],
        out_specs=pl.BlockSpec((blk, n), lambda i: (i, 0)),
        out_shape=jax.ShapeDtypeStruct((m, n), jnp.float32),
    )(x1, x2, w1, w2, b.reshape(1, n))


def _gcu_ab_kernel(raw_ref, feat_ref, wp_ref, bp_ref, wt_ref, wb_ref, bc_ref,
                   *o_refs):
    pf = jnp.dot(raw_ref[...], wp_ref[...], preferred_element_type=jnp.float32)
    pf = jnp.maximum(pf + bp_ref[...], 0.0)
    y = jnp.dot(pf, wt_ref[...], preferred_element_type=jnp.float32)
    y = y + jnp.dot(feat_ref[...], wb_ref[...], preferred_element_type=jnp.float32)
    y = y + bc_ref[...]
    for t, o in enumerate(o_refs):
        o[...] = y[:, t * 128:(t + 1) * 128]


def _gcu_ab(raw, feat, wp, bp, wtop, wbot, bcat):
    m = raw.shape[0]
    kr = raw.shape[1]
    kf = feat.shape[1]
    outs = pl.pallas_call(
        _gcu_ab_kernel,
        grid=(m // ROW_BLK,),
        in_specs=[
            pl.BlockSpec((ROW_BLK, kr), lambda i: (i, 0)),
            pl.BlockSpec((ROW_BLK, kf), lambda i: (i, 0)),
            pl.BlockSpec((kr, 64), lambda i: (0, 0)),
            pl.BlockSpec((1, 64), lambda i: (0, 0)),
            pl.BlockSpec((64, 512), lambda i: (0, 0)),
            pl.BlockSpec((kf, 512), lambda i: (0, 0)),
            pl.BlockSpec((1, 512), lambda i: (0, 0)),
        ],
        out_specs=[pl.BlockSpec((ROW_BLK, 128), lambda i: (i, 0))] * 4,
        out_shape=[jax.ShapeDtypeStruct((m, 128), jnp.float32)] * 4,
    )(raw, feat, wp, bp.reshape(1, 64), wtop, wbot, bcat.reshape(1, 512))
    return outs


def _edge_mm_kernel(g_ref, w_ref, b_ref, o_ref):
    y = jnp.dot(g_ref[...], w_ref[...], preferred_element_type=jnp.float32)
    o_ref[...] = jnp.maximum(y + b_ref[...], 0.0)


def _edge_mm(g, w, b):
    e = g.shape[0]
    return pl.pallas_call(
        _edge_mm_kernel,
        grid=(e // EDGE_BLK,),
        in_specs=[
            pl.BlockSpec((EDGE_BLK, 128), lambda i: (i, 0)),
            pl.BlockSpec((128, 128), lambda i: (0, 0)),
            pl.BlockSpec((1, 128), lambda i: (0, 0)),
        ],
        out_specs=pl.BlockSpec((EDGE_BLK, 128), lambda i: (i, 0)),
        out_shape=jax.ShapeDtypeStruct((e, 128), jnp.float32),
    )(g, w, b.reshape(1, 128))


def _mlt2_segmax_kernel(x_ref, w_ref, b_ref, batch_ref, o_ref, acc_ref):
    i = pl.program_id(0)

    @pl.when(i == 0)
    def _():
        acc_ref[...] = jnp.full_like(acc_ref, NEG_BIG)

    y = jnp.dot(x_ref[...], w_ref[...], preferred_element_type=jnp.float32)
    y = jnp.maximum(y + b_ref[...], 0.0)
    bids = batch_ref[...]
    for g in range(NGRAPH):
        m = (bids == g)
        colmax = jnp.max(jnp.where(m, y, NEG_BIG), axis=0)
        acc_ref[g, :] = jnp.maximum(acc_ref[g, :], colmax)

    @pl.when(i == pl.num_programs(0) - 1)
    def _():
        a = acc_ref[...]
        o_ref[...] = jnp.where(a <= NEG_BIG, 0.0, a)


def _mlt2_segmax(x, w, b, batch):
    m, k = x.shape
    n = w.shape[1]
    return pl.pallas_call(
        _mlt2_segmax_kernel,
        grid=(m // ROW_BLK,),
        in_specs=[
            pl.BlockSpec((ROW_BLK, k), lambda i: (i, 0)),
            pl.BlockSpec((k, n), lambda i: (0, 0)),
            pl.BlockSpec((1, n), lambda i: (0, 0)),
            pl.BlockSpec((ROW_BLK, 1), lambda i: (i, 0)),
        ],
        out_specs=pl.BlockSpec((8, n), lambda i: (0, 0)),
        out_shape=jax.ShapeDtypeStruct((8, n), jnp.float32),
        scratch_shapes=[pltpu.VMEM((8, n), jnp.float32)],
    )(x, w, b.reshape(1, n), batch.reshape(m, 1))


def _cls_kernel(x3_ref, xg4_ref, batch_ref, wt_ref, wb_ref, b1_ref,
                w2_ref, b2_ref, w3_ref, b3_ref, o_ref):
    xgw = jnp.dot(xg4_ref[...], wb_ref[...], preferred_element_type=jnp.float32)
    bids = batch_ref[...]
    onehot = (bids == lax.broadcasted_iota(jnp.int32, (1, 8), 1)).astype(jnp.float32)
    gathered = jnp.dot(onehot, xgw, preferred_element_type=jnp.float32)
    h = jnp.dot(x3_ref[...], wt_ref[...], preferred_element_type=jnp.float32)
    h = jnp.maximum(h + gathered + b1_ref[...], 0.0)
    h = jnp.dot(h, w2_ref[...], preferred_element_type=jnp.float32)
    h = jnp.maximum(h + b2_ref[...], 0.0)
    y = jnp.dot(h, w3_ref[...], preferred_element_type=jnp.float32)
    o_ref[...] = y + b3_ref[...]


def _cls_head(x3, xg4, batch, wtop, wbot, b1, w2, b2, w3, b3):
    m = x3.shape[0]
    nout = w3.shape[1]
    return pl.pallas_call(
        _cls_kernel,
        grid=(m // ROW_BLK,),
        in_specs=[
            pl.BlockSpec((ROW_BLK, 256), lambda i: (i, 0)),
            pl.BlockSpec((8, 1024), lambda i: (0, 0)),
            pl.BlockSpec((ROW_BLK, 1), lambda i: (i, 0)),
            pl.BlockSpec((256, 1024), lambda i: (0, 0)),
            pl.BlockSpec((1024, 1024), lambda i: (0, 0)),
            pl.BlockSpec((1, 1024), lambda i: (0, 0)),
            pl.BlockSpec((1024, 512), lambda i: (0, 0)),
            pl.BlockSpec((1, 512), lambda i: (0, 0)),
            pl.BlockSpec((512, nout), lambda i: (0, 0)),
            pl.BlockSpec((1, nout), lambda i: (0, 0)),
        ],
        out_specs=pl.BlockSpec((ROW_BLK, nout), lambda i: (i, 0)),
        out_shape=jax.ShapeDtypeStruct((m, nout), jnp.float32),
    )(x3, xg4, batch.reshape(m, 1), wtop, wbot, b1.reshape(1, 1024),
      w2, b2.reshape(1, 512), w3, b3.reshape(1, nout))


_NC, _NS = 2, 16
_NW = _NC * _NS
_EPW = N_EPAD // _NW
_GCH = 896
_GH = _GCH // 2
_DN = 313
_DN1 = _DN + 1
_NP = _NW * _DN
_SCH = 4000
_GB = 16
_CBUF = _SCH + _GB + 32
_CB2 = _SCH + _GB
_CAP = 168192
_NCHK = N_EDGES // _SCH


def _sc_mesh():
    return plsc.VectorSubcoreMesh(core_axis_name="c", subcore_axis_name="s")


_GNQ = 8
_GQ = _GCH // _GNQ


def _sc_gather_add(a_tab, b_tab, dst, src):
    @functools.partial(
        pl.kernel, mesh=_sc_mesh(),
        compiler_params=pltpu.CompilerParams(needs_layout_passes=False),
        out_type=jax.ShapeDtypeStruct((N_EPAD, 128), jnp.float32),
        scratch_types=[
            pltpu.VMEM((_GCH,), jnp.int32),
            pltpu.VMEM((_GCH,), jnp.int32),
            pltpu.VMEM((_GQ, 128), jnp.float32),
            pltpu.VMEM((_GQ, 128), jnp.float32),
            pltpu.VMEM((_GQ, 128), jnp.float32),
            pltpu.VMEM((_GQ, 128), jnp.float32),
            pltpu.SemaphoreType.DMA,
            pltpu.SemaphoreType.DMA,
            pltpu.SemaphoreType.DMA,
            pltpu.SemaphoreType.DMA,
            pltpu.SemaphoreType.DMA,
            pltpu.SemaphoreType.DMA,
        ])
    def k(a_hbm, b_hbm, dst_hbm, src_hbm, g_hbm,
          di_v, si_v, a0, a1, b0, b1, ga0, ga1, gb0, gb1, w0, w1):
        wid = lax.axis_index("s") * _NC + lax.axis_index("c")
        base = wid * _EPW
        abufs, bbufs = (a0, a1), (b0, b1)
        gas, gbs, ws = (ga0, ga1), (gb0, gb1), (w0, w1)

        def body(c, carry):
            off = base + c * _GCH
            pltpu.sync_copy(dst_hbm.at[pl.ds(off, _GCH)], di_v)
            pltpu.sync_copy(src_hbm.at[pl.ds(off, _GCH)], si_v)

            def startg(q):
                p = q % 2
                pltpu.async_copy(a_hbm.at[di_v.at[pl.ds(q * _GQ, _GQ)]],
                                 abufs[p], gas[p])
                pltpu.async_copy(b_hbm.at[si_v.at[pl.ds(q * _GQ, _GQ)]],
                                 bbufs[p], gbs[p])

            startg(0)
            startg(1)
            for q in range(_GNQ):
                p = q % 2
                pltpu.make_async_copy(a_hbm.at[di_v.at[pl.ds(q * _GQ, _GQ)]],
                                      abufs[p], gas[p]).wait()
                pltpu.make_async_copy(b_hbm.at[si_v.at[pl.ds(q * _GQ, _GQ)]],
                                      bbufs[p], gbs[p]).wait()

                def fuse(r, cc):
                    for kk in range(8):
                        sl = pl.ds(kk * 16, 16)
                        av = abufs[p][r, sl]
                        bv = bbufs[p][r, sl]
                        abufs[p][r, sl] = jnp.maximum(av + bv, 0.0)
                    return cc

                lax.fori_loop(0, _GQ, fuse, 0)
                pltpu.async_copy(abufs[p],
                                 g_hbm.at[pl.ds(off + q * _GQ, _GQ)], ws[p])
                if q + 2 < _GNQ:
                    pltpu.make_async_copy(
                        abufs[p], g_hbm.at[pl.ds(off + q * _GQ, _GQ)],
                        ws[p]).wait()
                    startg(q + 2)
            for q in (_GNQ - 2, _GNQ - 1):
                p = q % 2
                pltpu.make_async_copy(
                    abufs[p], g_hbm.at[pl.ds(off + q * _GQ, _GQ)],
                    ws[p]).wait()
            return carry

        lax.fori_loop(0, _EPW // _GCH, body, 0)

    return k(a_tab, b_tab, dst, src)


def _sc_compact(dst):
    @functools.partial(
        pl.kernel, mesh=_sc_mesh(),
        compiler_params=pltpu.CompilerParams(needs_layout_passes=False),
        out_type=[jax.ShapeDtypeStruct((_NW * _CAP,), jnp.int32),
                  jax.ShapeDtypeStruct((_NW * _CAP,), jnp.int32),
                  jax.ShapeDtypeStruct((_NW * 16,), jnp.int32)],
        scratch_types=[
            pltpu.VMEM((_SCH,), jnp.int32),
            pltpu.VMEM((_CBUF,), jnp.int32),
            pltpu.VMEM((_CBUF,), jnp.int32),
            pltpu.VMEM((_CBUF,), jnp.int32),
            pltpu.VMEM((_CBUF,), jnp.int32),
            pltpu.VMEM((16,), jnp.int32),
            pltpu.SemaphoreType.DMA,
            pltpu.SemaphoreType.DMA,
            pltpu.SemaphoreType.DMA,
            pltpu.SemaphoreType.DMA,
        ])
    def k(dst_hbm, ce_hbm, cl_hbm, cnt_hbm,
          dbuf, ce0, ce1, cl0, cl1, cbuf, se0, se1, sl0, sl1):
        wid = lax.axis_index("s") * _NC + lax.axis_index("c")
        lo = wid * _DN
        base = wid * _CAP
        lane = lax.iota(jnp.int32, 16)
        dn_splat = jnp.full((16,), _DN, jnp.int32)
        zero_splat = jnp.full((16,), 0, jnp.int32)
        cebufs, clbufs = (ce0, ce1), (cl0, cl1)
        esems, lsems = (se0, se1), (sl0, sl1)

        def one_chunk(c, u, cg):
            pltpu.sync_copy(dst_hbm.at[pl.ds(c * _SCH, _SCH)], dbuf)

            def scan(j, ptr):
                vd = dbuf[pl.ds(j * 16, 16)]
                m = (vd >= lo) & (vd < lo + _DN)
                mi = jnp.where(m, 1, 0)
                pos = plsc.cumsum(mi)
                eid = c * _SCH + j * 16 + lane
                tgt = jnp.where(m, ptr + pos - 1, _CBUF - 1)
                plsc.store_scatter(cebufs[u], [tgt], eid)
                plsc.store_scatter(clbufs[u], [tgt], vd - lo)
                return ptr + jnp.max(pos)

            total = lax.fori_loop(0, _SCH // 16, scan, 0)
            cebufs[u][pl.ds(total, 16)] = zero_splat
            clbufs[u][pl.ds(total, 16)] = dn_splat
            pltpu.async_copy(cebufs[u].at[pl.ds(0, _SCH)],
                             ce_hbm.at[pl.ds(base + cg * 8, _SCH)], esems[u])
            pltpu.async_copy(clbufs[u].at[pl.ds(0, _SCH)],
                             cl_hbm.at[pl.ds(base + cg * 8, _SCH)], lsems[u])
            return cg + (total + 7) // 8

        def body(i, cg):
            for u in range(2):
                @pl.when(i > 0)
                def _():
                    pltpu.make_async_copy(
                        cebufs[u].at[pl.ds(0, _SCH)],
                        ce_hbm.at[pl.ds(base, _SCH)], esems[u]).wait()
                    pltpu.make_async_copy(
                        clbufs[u].at[pl.ds(0, _SCH)],
                        cl_hbm.at[pl.ds(base, _SCH)], lsems[u]).wait()
                cg = one_chunk(2 * i + u, u, cg)
            return cg

        cum = lax.fori_loop(0, _NCHK // 2, body, 0) * 8
        for u in range(2):
            pltpu.make_async_copy(cebufs[u].at[pl.ds(0, _SCH)],
                                  ce_hbm.at[pl.ds(base, _SCH)], esems[u]).wait()
            pltpu.make_async_copy(clbufs[u].at[pl.ds(0, _SCH)],
                                  cl_hbm.at[pl.ds(base, _SCH)], lsems[u]).wait()
        cbuf[pl.ds(0, 16)] = zero_splat + cum
        pltpu.sync_copy(cbuf, cnt_hbm.at[pl.ds(wid * 16, 16)])

    return k(dst)


def _sc_segmax(h, comp):
    ce_all, cl_all, cnts = comp
    zeros_f = jnp.zeros((_DN1 * 128,), jnp.float32)

    @functools.partial(
        pl.kernel, mesh=_sc_mesh(),
        compiler_params=pltpu.CompilerParams(needs_layout_passes=False),
        out_type=jax.ShapeDtypeStruct((_NP * 128,), jnp.float32),
        scratch_types=[
            pltpu.VMEM((_DN1 * 128,), jnp.float32),
            pltpu.VMEM((_CB2,), jnp.int32),
            pltpu.VMEM((_CB2,), jnp.int32),
            pltpu.VMEM((16,), jnp.int32),
            pltpu.VMEM((_GB, 128), jnp.float32),
            pltpu.VMEM((_GB, 128), jnp.float32),
            pltpu.SemaphoreType.DMA,
            pltpu.SemaphoreType.DMA,
        ])
    def k(h_hbm, ce_hbm, cl_hbm, cnt_hbm, z_hbm, out_hbm,
          out_l, cebuf, clbuf, cbuf, hbuf0, hbuf1, sem0, sem1):
        wid = lax.axis_index("s") * _NC + lax.axis_index("c")
        lo = wid * _DN
        base = wid * _CAP
        lane = lax.iota(jnp.int32, 16)

        pltpu.sync_copy(z_hbm, out_l)
        pltpu.sync_copy(cnt_hbm.at[pl.ds(wid * 16, 16)], cbuf)
        cnt = jnp.max(cbuf[pl.ds(0, 16)])
        nch = (cnt + _SCH - 1) // _SCH

        def chunk(cix, c0):
            pltpu.sync_copy(ce_hbm.at[pl.ds(base + cix * _SCH, _SCH)],
                            cebuf.at[pl.ds(0, _SCH)])
            pltpu.sync_copy(cl_hbm.at[pl.ds(base + cix * _SCH, _SCH)],
                            clbuf.at[pl.ds(0, _SCH)])
            valid = jnp.minimum(cnt - cix * _SCH, _SCH)

            def mask(j, cc):
                idx = j * 16 + lane
                mm = idx < valid
                ce = cebuf[pl.ds(j * 16, 16)]
                cl = clbuf[pl.ds(j * 16, 16)]
                cebuf[pl.ds(j * 16, 16)] = jnp.where(mm, ce, 0)
                clbuf[pl.ds(j * 16, 16)] = jnp.where(mm, cl, _DN)
                return cc

            lax.fori_loop(0, _CB2 // 16, mask, 0)
            nsub = (valid + _GB - 1) // _GB

            def start(s, buf, sem):
                pltpu.async_copy(h_hbm.at[cebuf.at[pl.ds(s * _GB, _GB)]],
                                 buf, sem)

            def rmw(s, buf, sem):
                pltpu.make_async_copy(h_hbm.at[cebuf.at[pl.ds(s * _GB, _GB)]],
                                      buf, sem).wait()
                for gq in range(_GB // 16):
                    dvec = clbuf[pl.ds(s * _GB + gq * 16, 16)]
                    for i in range(16):
                        dd = jnp.max(jnp.where(lane == i, dvec, 0))
                        rbase = dd * 128
                        for kk in range(8):
                            sl = pl.ds(rbase + kk * 16, 16)
                            hv = buf[gq * 16 + i, pl.ds(kk * 16, 16)]
                            out_l[sl] = jnp.maximum(out_l[sl], hv)

            @pl.when(nsub > 0)
            def _():
                start(0, hbuf0, sem0)

            def sub2(i, c1):
                s0 = 2 * i

                @pl.when(s0 + 1 < nsub)
                def _():
                    start(s0 + 1, hbuf1, sem1)

                rmw(s0, hbuf0, sem0)

                @pl.when(s0 + 2 < nsub)
                def _():
                    start(s0 + 2, hbuf0, sem0)

                @pl.when(s0 + 1 < nsub)
                def _():
                    rmw(s0 + 1, hbuf1, sem1)

                return c1

            lax.fori_loop(0, (nsub + 1) // 2, sub2, 0)
            return c0

        lax.fori_loop(0, nch, chunk, 0)
        pltpu.sync_copy(out_l.at[pl.ds(0, _DN * 128)],
                        out_hbm.at[pl.ds(lo * 128, _DN * 128)])

    out = k(h, ce_all, cl_all, cnts, zeros_f)
    return out.reshape(_NP, 128)[:N_NODES]


def _edge_conv(a_tab, b_tab, w2, b2, dst, src, comp):
    g = _sc_gather_add(a_tab, b_tab, dst, src)
    h = _edge_mm(g, w2, b2)
    return _sc_segmax(h, comp)


def _split_edge_params(gcu_p, xdim):
    out = {}
    for et in ('tpl', 'geo'):
        (w1, b1), (w2, b2) = gcu_p[et]
        w1a, w1b = w1[:xdim], w1[xdim:]
        out[et] = (w1a - w1b, w1b, b1, w2, b2)
    return out


def _gcu_layer(gcu_p, raw, feat, dst_t, src_t, dst_g, src_g, comp_t, comp_g):
    xdim = 64 + feat.shape[1]
    ep = _split_edge_params(gcu_p, xdim)
    (wp, bp), = gcu_p['pos_mlp']
    wtop = jnp.concatenate([ep['tpl'][0][:64], ep['tpl'][1][:64],
                            ep['geo'][0][:64], ep['geo'][1][:64]], axis=1)
    wbot = jnp.concatenate([ep['tpl'][0][64:], ep['tpl'][1][64:],
                            ep['geo'][0][64:], ep['geo'][1][64:]], axis=1)
    zeros = jnp.zeros_like(ep['tpl'][2])
    bcat = jnp.concatenate([ep['tpl'][2], zeros, ep['geo'][2], zeros])
    a_t, b_t, a_g, b_g = _gcu_ab(raw, feat, wp, bp, wtop, wbot, bcat)
    xt = _edge_conv(a_t, b_t, ep['tpl'][3], ep['tpl'][4], dst_t, src_t, comp_t)
    xg = _edge_conv(a_g, b_g, ep['geo'][3], ep['geo'][4], dst_g, src_g, comp_g)
    (wm, bm), = gcu_p['mlp']
    return _mm2(xt, xg, wm[:128], wm[128:], bm)


def kernel(pos, skin_input, motion, tpl_edge_index, geo_edge_index, batch, params):
    raw = jnp.concatenate([pos, skin_input[:, :8 * NB]], axis=1)
    pad = (jnp.arange(N_EPAD - N_EDGES, dtype=jnp.int32) * 13) % N_NODES
    def _p(v):
        return jnp.concatenate([v, pad])
    src_t, dst_t = _p(tpl_edge_index[0]), _p(tpl_edge_index[1])
    src_g, dst_g = _p(geo_edge_index[0]), _p(geo_edge_index[1])
    comp_t = _sc_compact(dst_t)
    comp_g = _sc_compact(dst_g)

    x1 = _gcu_layer(params['gcu1'], raw, motion, dst_t, src_t, dst_g, src_g,
                    comp_t, comp_g)

    (wl1, bl1), (wl2, bl2) = params['mlt2']
    h = _mm(x1, wl1, bl1)
    xg4 = _mlt2_segmax(h, wl2, bl2, batch)

    x2 = _gcu_layer(params['gcu2'], raw, x1, dst_t, src_t, dst_g, src_g,
                    comp_t, comp_g)
    x3 = _gcu_layer(params['gcu3'], raw, x2, dst_t, src_t, dst_g, src_g,
                    comp_t, comp_g)

    (wc1, bc1), (wc2, bc2) = params['cls_mlp']
    wo, bo = params['cls_out']
    return _cls_head(x3, xg4, batch, wc1[:256], wc1[256:], bc1,
                     wc2, bc2, wo, bo)
